# baseline (speedup 1.0000x reference)
import functools

import jax
import jax.numpy as jnp
from jax import lax
from jax.experimental import pallas as pl
from jax.experimental.pallas import tpu as pltpu
from jax.experimental.pallas import tpu_sc as plsc

L = 16
NC = 2
NS = 16
NW = NC * NS
NROW = 16384
NCOL = 200
ROWS_W = NROW // NW
RCHUNK = 32
NCHUNK = ROWS_W // RCHUNK
NFULL = NCOL // L
TAIL = NCOL - L
NACC = 4

_mesh = plsc.VectorSubcoreMesh(core_axis_name="c", subcore_axis_name="s")


@functools.partial(
    pl.kernel,
    mesh=_mesh,
    out_type=jax.ShapeDtypeStruct((NW, L), jnp.float32),
    compiler_params=pltpu.CompilerParams(
        needs_layout_passes=False, use_tc_tiling_on_sc=True
    ),
    scratch_types=[
        pltpu.VMEM((32,), jnp.float32),
        pltpu.VMEM((2, RCHUNK, NCOL), jnp.float32),
        pltpu.VMEM((2, RCHUNK, NCOL), jnp.int32),
        pltpu.VMEM((L,), jnp.float32),
        pltpu.SemaphoreType.DMA,
        pltpu.SemaphoreType.DMA,
    ],
)
def _wae_sc(x_hbm, t_hbm, table_hbm, out_hbm, table_v, xb, tb, pv, sem0, sem1):
    wid = lax.axis_index("s") * NC + lax.axis_index("c")
    base = wid * ROWS_W
    sems = (sem0, sem1)

    pltpu.sync_copy(table_hbm, table_v)
    tail_keep = lax.iota(jnp.int32, L) >= (L - (NCOL - NFULL * L))

    def start(c):
        b = c % 2
        src = pl.ds(base + c * RCHUNK, RCHUNK)
        return (
            pltpu.async_copy(x_hbm.at[src], xb.at[b], sems[b]),
            pltpu.async_copy(t_hbm.at[src], tb.at[b], sems[b]),
        )

    inflight = start(0)
    accs = (jnp.zeros((L,), jnp.float32),) * NACC
    for c in range(NCHUNK):
        for h in inflight:
            h.wait()
        if c + 1 < NCHUNK:
            inflight = start(c + 1)
        b = c % 2

        def body(r, a):
            a = list(a)
            for j in range(NFULL + 1):
                s = pl.ds(j * L if j < NFULL else TAIL, L)
                xv = xb[b, r, s]
                tv = tb[b, r, s]
                w = plsc.load_gather(table_v, [tv])
                wd = w * jnp.abs(xv - tv.astype(jnp.float32))
                if j == NFULL:
                    wd = jnp.where(tail_keep, wd, 0.0)
                a[j % NACC] = a[j % NACC] + wd
            return tuple(a)

        accs = plsc.parallel_loop(0, RCHUNK, 1, unroll=2, carry=accs)(body)

    pv[...] = accs[0] + accs[1] + accs[2] + accs[3]
    pltpu.sync_copy(pv, out_hbm.at[wid])


def kernel(inputs, targets, class_weights):
    m = jnp.sum(class_weights > 0).astype(jnp.float32)
    C = jnp.where(m > 0, 1.0 / m, 1.0)
    table = jnp.pad(class_weights * C, (0, 32 - class_weights.shape[0]))
    partials = _wae_sc(inputs, targets.astype(jnp.int32), table)
    return jnp.sum(partials)


# speedup vs baseline: 2.0900x; 2.0900x over previous
import functools

import jax
import jax.numpy as jnp
from jax import lax
from jax.experimental import pallas as pl
from jax.experimental.pallas import tpu as pltpu
from jax.experimental.pallas import tpu_sc as plsc

L = 16
NC = 2
NS = 16
NW = NC * NS
NROW = 200
NCOL = 16384
COLS_W = NCOL // NW
CCHUNK = 128
NCHUNK = COLS_W // CCHUNK
NVEC = CCHUNK // L
NACC = 4

_mesh = plsc.VectorSubcoreMesh(core_axis_name="c", subcore_axis_name="s")


@functools.partial(
    pl.kernel,
    mesh=_mesh,
    out_type=jax.ShapeDtypeStruct((NW, L), jnp.float32),
    compiler_params=pltpu.CompilerParams(needs_layout_passes=False),
    scratch_types=[
        pltpu.VMEM((32,), jnp.float32),
        pltpu.VMEM((2, NROW, CCHUNK), jnp.float32),
        pltpu.VMEM((2, NROW, CCHUNK), jnp.int32),
        pltpu.VMEM((L,), jnp.float32),
        pltpu.SemaphoreType.DMA,
        pltpu.SemaphoreType.DMA,
    ],
)
def _wae_sc(x_hbm, t_hbm, table_hbm, out_hbm, table_v, xb, tb, pv, sem0, sem1):
    wid = lax.axis_index("s") * NC + lax.axis_index("c")
    base = wid * COLS_W
    sems = (sem0, sem1)

    pltpu.sync_copy(table_hbm, table_v)

    def start(c):
        b = c % 2
        src = pl.ds(base + c * CCHUNK, CCHUNK)
        return (
            pltpu.async_copy(x_hbm.at[:, src], xb.at[b], sems[b]),
            pltpu.async_copy(t_hbm.at[:, src], tb.at[b], sems[b]),
        )

    inflight = start(0)
    accs = (jnp.zeros((L,), jnp.float32),) * NACC
    for c in range(NCHUNK):
        for h in inflight:
            h.wait()
        if c + 1 < NCHUNK:
            inflight = start(c + 1)
        b = c % 2

        def body(r, a):
            a = list(a)
            for j in range(NVEC):
                s = pl.ds(j * L, L)
                xv = xb[b, r, s]
                tv = tb[b, r, s]
                w = plsc.load_gather(table_v, [tv])
                wd = w * jnp.abs(xv - tv.astype(jnp.float32))
                a[j % NACC] = a[j % NACC] + wd
            return tuple(a)

        accs = plsc.parallel_loop(0, NROW, 1, unroll=2, carry=accs)(body)

    pv[...] = accs[0] + accs[1] + accs[2] + accs[3]
    pltpu.sync_copy(pv, out_hbm.at[wid])


def kernel(inputs, targets, class_weights):
    m = jnp.sum(class_weights > 0).astype(jnp.float32)
    C = jnp.where(m > 0, 1.0 / m, 1.0)
    table = jnp.pad(class_weights * C, (0, 32 - class_weights.shape[0]))
    partials = _wae_sc(inputs.T, targets.astype(jnp.int32).T, table)
    return jnp.sum(partials)
